# Initial kernel scaffold; baseline (speedup 1.0000x reference)
import jax
import jax.numpy as jnp
from jax.experimental import pallas as pl


def kernel(x, Wa, W0, b0, W1, b1, W2, b2):
    raise NotImplementedError("write your pallas kernel here")


# speedup vs baseline: 182.0542x; 182.0542x over previous
import functools

import jax
import jax.numpy as jnp
from jax.experimental import pallas as pl
from jax.experimental.pallas import tpu as pltpu

N = 512
LPAD = 131072


def _gcn_kernel(x_ref, wa_ref, w0_ref, b0_ref, w1_ref, b1_ref, w2_ref, b2_ref,
                o_ref, au_scr):
    col = jax.lax.broadcasted_iota(jnp.int32, (1, N), 1)

    def body(i, _):
        s = i * (N - 2) - (i * (i - 1)) // 2
        base = pl.multiple_of((s // 128) * 128, 128)
        r = s - base
        win = wa_ref[0, pl.ds(base, N + 128)].reshape(1, N + 128)
        vals = pltpu.roll(win, (N + 128) - r, axis=1)[:, :N]
        z = jnp.exp2(vals * jnp.float32(-1.4426950408889634))
        sig = pl.reciprocal(1.0 + z, approx=True)
        row = jnp.where(col > i, sig, 0.0)
        au_scr[pl.ds(i, 1), :] = row
        return 0

    jax.lax.fori_loop(0, N, body, 0)

    au = au_scr[:, :]
    eye = jnp.where(
        jax.lax.broadcasted_iota(jnp.int32, (N, N), 0)
        == jax.lax.broadcasted_iota(jnp.int32, (N, N), 1),
        1.0, 0.0)
    a = au + au.T + eye

    deg = jnp.sum(a, axis=0, keepdims=True)
    dinv = jax.lax.rsqrt(deg)
    ahat = a * dinv * dinv.T

    def dot_w(a, w):
        return jnp.dot(a.astype(jnp.bfloat16), w.astype(jnp.bfloat16),
                       preferred_element_type=jnp.float32)

    dot_a = functools.partial(jnp.dot, preferred_element_type=jnp.float32,
                              precision=jax.lax.Precision.HIGHEST)
    h = dot_w(x_ref[:, :], w0_ref[:, :])
    h = jax.nn.relu(dot_a(ahat, h) + b0_ref[0, :])
    h = dot_w(h, w1_ref[:, :])
    h = jax.nn.relu(dot_a(ahat, h) + b1_ref[0, :])
    h = dot_w(h, w2_ref[:, :])
    o_ref[:, :] = jax.nn.relu(dot_a(ahat, h) + b2_ref[0, :])


@jax.jit
def kernel(x, Wa, W0, b0, W1, b1, W2, b2):
    wa_pad = jnp.zeros((1, LPAD), jnp.float32).at[0, 1:1 + N * (N - 1) // 2].set(
        Wa.reshape(-1))
    return pl.pallas_call(
        _gcn_kernel,
        out_shape=jax.ShapeDtypeStruct((N, W2.shape[1]), jnp.float32),
        scratch_shapes=[pltpu.VMEM((N, N), jnp.float32)],
    )(x, wa_pad, W0, b0.reshape(1, -1), W1, b1.reshape(1, -1),
      W2, b2.reshape(1, -1))


# speedup vs baseline: 197.5223x; 1.0850x over previous
import functools

import jax
import jax.numpy as jnp
from jax.experimental import pallas as pl
from jax.experimental.pallas import tpu as pltpu

N = 512
LPAD = 131072


def _gcn_kernel(x_ref, wa_ref, w0_ref, b0_ref, w1_ref, b1_ref, w2_ref, b2_ref,
                o_ref, au_scr):
    def body(i, _):
        s = i * (N - 2) - (i * (i - 1)) // 2
        base = pl.multiple_of((s // 128) * 128, 128)
        r = s - base
        win = wa_ref[0, pl.ds(base, N + 128)].reshape(1, N + 128)
        au_scr[pl.ds(i, 1), :] = pltpu.roll(win, (N + 128) - r, axis=1)[:, :N]
        return 0

    jax.lax.fori_loop(0, N, body, 0)

    rows = jax.lax.broadcasted_iota(jnp.int32, (N, N), 0)
    cols = jax.lax.broadcasted_iota(jnp.int32, (N, N), 1)
    z = jnp.exp2(au_scr[:, :] * jnp.float32(-1.4426950408889634))
    au = jnp.where(cols > rows, pl.reciprocal(1.0 + z, approx=True), 0.0)
    eye = jnp.where(rows == cols, 1.0, 0.0)
    a = au + au.T + eye

    deg = jnp.sum(a, axis=0, keepdims=True)
    dinv = jax.lax.rsqrt(deg)
    ahat = a * dinv * dinv.T

    def dot_w(a, w):
        return jnp.dot(a.astype(jnp.bfloat16), w.astype(jnp.bfloat16),
                       preferred_element_type=jnp.float32)

    dot_a = functools.partial(jnp.dot, preferred_element_type=jnp.float32,
                              precision=jax.lax.Precision.HIGHEST)
    h = dot_w(x_ref[:, :], w0_ref[:, :])
    h = jax.nn.relu(dot_a(ahat, h) + b0_ref[0, :])
    h = dot_w(h, w1_ref[:, :])
    h = jax.nn.relu(dot_a(ahat, h) + b1_ref[0, :])
    h = dot_w(h, w2_ref[:, :])
    o_ref[:, :] = jax.nn.relu(dot_a(ahat, h) + b2_ref[0, :])


@jax.jit
def kernel(x, Wa, W0, b0, W1, b1, W2, b2):
    wa_pad = jnp.zeros((1, LPAD), jnp.float32).at[0, 1:1 + N * (N - 1) // 2].set(
        Wa.reshape(-1))
    return pl.pallas_call(
        _gcn_kernel,
        out_shape=jax.ShapeDtypeStruct((N, W2.shape[1]), jnp.float32),
        scratch_shapes=[pltpu.VMEM((N, N), jnp.float32)],
    )(x, wa_pad, W0, b0.reshape(1, -1), W1, b1.reshape(1, -1),
      W2, b2.reshape(1, -1))


# speedup vs baseline: 426.5204x; 2.1594x over previous
import functools

import jax
import jax.numpy as jnp
from jax.experimental import pallas as pl
from jax.experimental.pallas import tpu as pltpu

N = 512
LPAD = 131072


def _gcn_kernel(x_ref, wa_ref, w0_ref, b0_ref, w1_ref, b1_ref, w2_ref, b2_ref,
                o_ref, au_scr):
    def body(i8, _):
        for t in range(8):
            i = i8 * 8 + t
            s = i * (N - 2) - (i * (i - 1)) // 2
            base = pl.multiple_of((s // 128) * 128, 128)
            r = s - base
            win = wa_ref[0, pl.ds(base, N + 128)].reshape(1, N + 128)
            au_scr[pl.ds(i, 1), :] = pltpu.roll(win, (N + 128) - r, axis=1)[:, :N]
        return 0

    jax.lax.fori_loop(0, N // 8, body, 0)

    rows = jax.lax.broadcasted_iota(jnp.int32, (N, N), 0)
    cols = jax.lax.broadcasted_iota(jnp.int32, (N, N), 1)
    z = jnp.exp2(au_scr[:, :] * jnp.float32(-1.4426950408889634))
    au = jnp.where(cols > rows, pl.reciprocal(1.0 + z, approx=True), 0.0)
    eye = jnp.where(rows == cols, 1.0, 0.0)
    a = au + au.T + eye

    deg = jnp.sum(a, axis=0, keepdims=True)
    dinv = jax.lax.rsqrt(deg)
    ahat = a * dinv * dinv.T

    def dot_w(a, w):
        return jnp.dot(a.astype(jnp.bfloat16), w.astype(jnp.bfloat16),
                       preferred_element_type=jnp.float32)

    dot_a = functools.partial(jnp.dot, preferred_element_type=jnp.float32,
                              precision=jax.lax.Precision.HIGHEST)
    h = dot_w(x_ref[:, :], w0_ref[:, :])
    h = jax.nn.relu(dot_a(ahat, h) + b0_ref[0, :])
    h = dot_w(h, w1_ref[:, :])
    h = jax.nn.relu(dot_a(ahat, h) + b1_ref[0, :])
    h = dot_w(h, w2_ref[:, :])
    o_ref[:, :] = jax.nn.relu(dot_a(ahat, h) + b2_ref[0, :])


@jax.jit
def kernel(x, Wa, W0, b0, W1, b1, W2, b2):
    wa_pad = jnp.zeros((1, LPAD), jnp.float32).at[0, 1:1 + N * (N - 1) // 2].set(
        Wa.reshape(-1))
    return pl.pallas_call(
        _gcn_kernel,
        out_shape=jax.ShapeDtypeStruct((N, W2.shape[1]), jnp.float32),
        scratch_shapes=[pltpu.VMEM((N, N), jnp.float32)],
    )(x, wa_pad, W0, b0.reshape(1, -1), W1, b1.reshape(1, -1),
      W2, b2.reshape(1, -1))
